# baseline (speedup 1.0000x reference)
import functools

import jax
import jax.numpy as jnp
from jax import lax
from jax.experimental import pallas as pl
from jax.experimental.pallas import tpu as pltpu
from jax.experimental.pallas import tpu_sc as plsc

N, D, E, DFF = 8192, 768, 64, 1536
DP = D // 2
CAP = 160
NSLOT = E * CAP
TRASH = NSLOT
B = 512
NB = N // B
NW = 32
TPW = N // NW
CH = 128
NCH = TPW // CH

_HI = -65536


def _pack_bf16_pair(a_f32, b_f32):
    ai = lax.bitcast_convert_type(a_f32, jnp.int32)
    bi = lax.bitcast_convert_type(b_f32, jnp.int32)
    pa = lax.shift_right_logical(ai + 0x8000, 16)
    pb = (bi + 0x8000) & _HI
    return pa | pb


def _unpack_lo(p):
    return lax.bitcast_convert_type(p << 16, jnp.float32)


def _unpack_hi(p):
    return lax.bitcast_convert_type(p & _HI, jnp.float32)


def _router_body(x_ref, wr_ref, scat_ref, gath_ref, scale_ref, xp_ref, counts_ref):
    step = pl.program_id(0)

    @pl.when(step == 0)
    def _():
        counts_ref[...] = jnp.zeros_like(counts_ref)

    x = x_ref[...]
    xp_ref[...] = _pack_bf16_pair(x[:, :DP], x[:, DP:])

    logits = jnp.dot(x, wr_ref[...], preferred_element_type=jnp.float32)
    m = jnp.max(logits, axis=1, keepdims=True)
    ssum = jnp.sum(jnp.exp(logits - m), axis=1, keepdims=True)
    gate = 1.0 / ssum

    lane = lax.broadcasted_iota(jnp.int32, (B, E), 1)
    e_idx = jnp.min(jnp.where(logits == m, lane, E), axis=1, keepdims=True)
    onehot = (lane == e_idx).astype(jnp.bfloat16)

    r_io = lax.broadcasted_iota(jnp.int32, (B, B), 0)
    c_io = lax.broadcasted_iota(jnp.int32, (B, B), 1)
    tril = (r_io >= c_io).astype(jnp.bfloat16)
    csum = jnp.dot(tril, onehot, preferred_element_type=jnp.float32)
    onehot_f = onehot.astype(jnp.float32)
    rank_incl = jnp.sum(csum * onehot_f, axis=1, keepdims=True)
    base = jnp.sum(counts_ref[...] * onehot_f, axis=1, keepdims=True)
    counts_ref[...] = counts_ref[...] + csum[B - 1 : B, :]

    pos = base + rank_incl - 1.0
    keep = pos < CAP
    clipped = jnp.minimum(pos, CAP - 1.0).astype(jnp.int32)
    gath = e_idx * CAP + clipped
    scat = jnp.where(keep, gath, TRASH)

    scat_ref[...] = scat
    gath_ref[...] = gath
    scale_ref[...] = jnp.where(keep, gate, 0.0)


def _router(x, w_router):
    return pl.pallas_call(
        _router_body,
        grid=(NB,),
        in_specs=[
            pl.BlockSpec((B, D), lambda i: (i, 0)),
            pl.BlockSpec((D, E), lambda i: (0, 0)),
        ],
        out_specs=[
            pl.BlockSpec((B, 1), lambda i: (i, 0)),
            pl.BlockSpec((B, 1), lambda i: (i, 0)),
            pl.BlockSpec((B, 1), lambda i: (i, 0)),
            pl.BlockSpec((B, DP), lambda i: (i, 0)),
        ],
        out_shape=[
            jax.ShapeDtypeStruct((N, 1), jnp.int32),
            jax.ShapeDtypeStruct((N, 1), jnp.int32),
            jax.ShapeDtypeStruct((N, 1), jnp.float32),
            jax.ShapeDtypeStruct((N, DP), jnp.int32),
        ],
        scratch_shapes=[pltpu.VMEM((1, E), jnp.float32)],
        compiler_params=pltpu.CompilerParams(
            dimension_semantics=("arbitrary",)
        ),
    )(x, w_router)


@functools.cache
def _sc_mesh():
    return plsc.VectorSubcoreMesh(
        core_axis_name="c", subcore_axis_name="s", num_cores=2, num_subcores=16
    )


@functools.cache
def _make_dispatch():
    @functools.partial(
        pl.kernel,
        out_type=jax.ShapeDtypeStruct((NSLOT + 8, DP), jnp.int32),
        mesh=_sc_mesh(),
        scratch_types=[
            pltpu.VMEM((CH,), jnp.int32),
            pltpu.VMEM((CH, DP), jnp.int32),
            pltpu.SemaphoreType.DMA,
        ],
    )
    def _dispatch(xp_hbm, idx_hbm, disp_hbm, idx_v, xbuf, sem):
        wid = lax.axis_index("s") * 2 + lax.axis_index("c")
        for j in range(NCH):
            pltpu.sync_copy(idx_hbm.at[wid * NCH + j], idx_v)
            pltpu.sync_copy(xp_hbm.at[pl.ds(wid * TPW + j * CH, CH)], xbuf)
            pltpu.async_copy(xbuf, disp_hbm.at[idx_v], sem).wait()

    return _dispatch


def _ffn_body(d_ref, wi_ref, wo_ref, y_ref):
    d = d_ref[...]
    da = _unpack_lo(d)
    db = _unpack_hi(d)
    wi = wi_ref[0]
    h = jnp.maximum(
        jnp.dot(da, wi[:DP], preferred_element_type=jnp.float32)
        + jnp.dot(db, wi[DP:], preferred_element_type=jnp.float32),
        0.0,
    )
    y = jnp.dot(h, wo_ref[0], preferred_element_type=jnp.float32)
    y_ref[...] = _pack_bf16_pair(y[:, :DP], y[:, DP:])


def _ffn(disp, w_in, w_out):
    return pl.pallas_call(
        _ffn_body,
        grid=(E,),
        in_specs=[
            pl.BlockSpec((CAP, DP), lambda e: (e, 0)),
            pl.BlockSpec((1, D, DFF), lambda e: (e, 0, 0)),
            pl.BlockSpec((1, DFF, D), lambda e: (e, 0, 0)),
        ],
        out_specs=pl.BlockSpec((CAP, DP), lambda e: (e, 0)),
        out_shape=jax.ShapeDtypeStruct((NSLOT, DP), jnp.int32),
        compiler_params=pltpu.CompilerParams(
            dimension_semantics=("parallel",)
        ),
    )(disp, w_in, w_out)


@functools.cache
def _make_combine():
    @functools.partial(
        pl.kernel,
        out_type=jax.ShapeDtypeStruct((N, DP), jnp.int32),
        mesh=_sc_mesh(),
        scratch_types=[
            pltpu.VMEM((CH,), jnp.int32),
            pltpu.VMEM((CH, DP), jnp.int32),
            pltpu.SemaphoreType.DMA,
        ],
    )
    def _combine(y_hbm, idx_hbm, out_hbm, idx_v, ybuf, sem):
        wid = lax.axis_index("s") * 2 + lax.axis_index("c")
        for j in range(NCH):
            pltpu.sync_copy(idx_hbm.at[wid * NCH + j], idx_v)
            pltpu.async_copy(y_hbm.at[idx_v], ybuf, sem).wait()
            pltpu.sync_copy(ybuf, out_hbm.at[pl.ds(wid * TPW + j * CH, CH)])

    return _combine


def _finish_body(g_ref, s_ref, o_ref):
    g = g_ref[...]
    s = s_ref[...]
    o_ref[:, :DP] = _unpack_lo(g) * s
    o_ref[:, DP:] = _unpack_hi(g) * s


def _finish(gathered, scale):
    return pl.pallas_call(
        _finish_body,
        grid=(NB,),
        in_specs=[
            pl.BlockSpec((B, DP), lambda i: (i, 0)),
            pl.BlockSpec((B, 1), lambda i: (i, 0)),
        ],
        out_specs=pl.BlockSpec((B, D), lambda i: (i, 0)),
        out_shape=jax.ShapeDtypeStruct((N, D), jnp.float32),
    )(gathered, scale)


def kernel(x, w_router, w_in, w_out):
    scat, gath, scale, xp = _router(x, w_router)
    scat = scat.reshape(NW * NCH, CH)
    gath = gath.reshape(NW * NCH, CH)
    disp = _make_dispatch()(xp, scat)
    y = _ffn(disp, w_in, w_out)
    gathered = _make_combine()(y, gath)
    return _finish(gathered, scale)


# speedup vs baseline: 1.0431x; 1.0431x over previous
import functools

import jax
import jax.numpy as jnp
from jax import lax
from jax.experimental import pallas as pl
from jax.experimental.pallas import tpu as pltpu
from jax.experimental.pallas import tpu_sc as plsc

N, D, E, DFF = 8192, 768, 64, 1536
DP = D // 2
CAP = 160
NSLOT = E * CAP
TRASH = NSLOT
B = 512
NB = N // B
NW = 32
TPW = N // NW
CH = 128
NCH = TPW // CH

_HI = -65536


def _pack_bf16_pair(a_f32, b_f32):
    ai = lax.bitcast_convert_type(a_f32, jnp.int32)
    bi = lax.bitcast_convert_type(b_f32, jnp.int32)
    pa = lax.shift_right_logical(ai + 0x8000, 16)
    pb = (bi + 0x8000) & _HI
    return pa | pb


def _unpack_lo(p):
    return lax.bitcast_convert_type(p << 16, jnp.float32)


def _unpack_hi(p):
    return lax.bitcast_convert_type(p & _HI, jnp.float32)


def _router_body(x_ref, wr_ref, scat_ref, gath_ref, scale_ref, xp_ref, counts_ref):
    step = pl.program_id(0)

    @pl.when(step == 0)
    def _():
        counts_ref[...] = jnp.zeros_like(counts_ref)

    x = x_ref[...]
    xp_ref[...] = _pack_bf16_pair(x[:, :DP], x[:, DP:])

    logitsT = lax.dot_general(
        wr_ref[...], x, (((0,), (1,)), ((), ())),
        preferred_element_type=jnp.float32,
    )
    m = jnp.max(logitsT, axis=0, keepdims=True)
    ssum = jnp.sum(jnp.exp(logitsT - m), axis=0, keepdims=True)
    gate = 1.0 / ssum

    row = lax.broadcasted_iota(jnp.int32, (E, B), 0)
    e_idx = jnp.min(jnp.where(logitsT == m, row, E), axis=0, keepdims=True)
    onehotT = (row == e_idx).astype(jnp.bfloat16)

    r_io = lax.broadcasted_iota(jnp.int32, (B, B), 0)
    c_io = lax.broadcasted_iota(jnp.int32, (B, B), 1)
    triu = (r_io <= c_io).astype(jnp.bfloat16)
    csumT = jnp.dot(onehotT, triu, preferred_element_type=jnp.float32)
    onehot_f = onehotT.astype(jnp.float32)
    rank_incl = jnp.sum(csumT * onehot_f, axis=0, keepdims=True)
    base = jnp.dot(counts_ref[...], onehot_f, preferred_element_type=jnp.float32)
    counts_ref[...] = counts_ref[...] + csumT[:, B - 1 : B].reshape(1, E)

    pos = base + rank_incl - 1.0
    keep = pos < CAP
    clipped = jnp.minimum(pos, CAP - 1.0).astype(jnp.int32)
    gath = e_idx * CAP + clipped
    scat = jnp.where(keep, gath, TRASH)

    scat_ref[...] = scat.reshape(1, 1, B)
    gath_ref[...] = gath.reshape(1, 1, B)
    scale_ref[...] = jnp.where(keep, gate, 0.0).reshape(1, 1, B)


def _router(x, w_router):
    return pl.pallas_call(
        _router_body,
        grid=(NB,),
        in_specs=[
            pl.BlockSpec((B, D), lambda i: (i, 0)),
            pl.BlockSpec((D, E), lambda i: (0, 0)),
        ],
        out_specs=[
            pl.BlockSpec((1, 1, B), lambda i: (i, 0, 0)),
            pl.BlockSpec((1, 1, B), lambda i: (i, 0, 0)),
            pl.BlockSpec((1, 1, B), lambda i: (i, 0, 0)),
            pl.BlockSpec((B, DP), lambda i: (i, 0)),
        ],
        out_shape=[
            jax.ShapeDtypeStruct((NB, 1, B), jnp.int32),
            jax.ShapeDtypeStruct((NB, 1, B), jnp.int32),
            jax.ShapeDtypeStruct((NB, 1, B), jnp.float32),
            jax.ShapeDtypeStruct((N, DP), jnp.int32),
        ],
        scratch_shapes=[pltpu.VMEM((1, E), jnp.float32)],
        compiler_params=pltpu.CompilerParams(
            dimension_semantics=("arbitrary",)
        ),
    )(x, w_router)


@functools.cache
def _sc_mesh():
    return plsc.VectorSubcoreMesh(
        core_axis_name="c", subcore_axis_name="s", num_cores=2, num_subcores=16
    )


@functools.cache
def _make_dispatch():
    @functools.partial(
        pl.kernel,
        out_type=jax.ShapeDtypeStruct((NSLOT + 8, DP), jnp.int32),
        mesh=_sc_mesh(),
        scratch_types=[
            pltpu.VMEM((CH,), jnp.int32),
            pltpu.VMEM((CH, DP), jnp.int32),
            pltpu.SemaphoreType.DMA,
        ],
    )
    def _dispatch(xp_hbm, idx_hbm, disp_hbm, idx_v, xbuf, sem):
        wid = lax.axis_index("s") * 2 + lax.axis_index("c")
        for j in range(NCH):
            pltpu.sync_copy(idx_hbm.at[wid * NCH + j], idx_v)
            pltpu.sync_copy(xp_hbm.at[pl.ds(wid * TPW + j * CH, CH)], xbuf)
            pltpu.async_copy(xbuf, disp_hbm.at[idx_v], sem).wait()

    return _dispatch


def _ffn_body(d_ref, wi_ref, wo_ref, y_ref):
    d = d_ref[...]
    da = _unpack_lo(d)
    db = _unpack_hi(d)
    wi = wi_ref[0]
    h = jnp.maximum(
        jnp.dot(da, wi[:DP], preferred_element_type=jnp.float32)
        + jnp.dot(db, wi[DP:], preferred_element_type=jnp.float32),
        0.0,
    )
    y = jnp.dot(h, wo_ref[0], preferred_element_type=jnp.float32)
    y_ref[...] = _pack_bf16_pair(y[:, :DP], y[:, DP:])


def _ffn(disp, w_in, w_out):
    return pl.pallas_call(
        _ffn_body,
        grid=(E,),
        in_specs=[
            pl.BlockSpec((CAP, DP), lambda e: (e, 0)),
            pl.BlockSpec((1, D, DFF), lambda e: (e, 0, 0)),
            pl.BlockSpec((1, DFF, D), lambda e: (e, 0, 0)),
        ],
        out_specs=pl.BlockSpec((CAP, DP), lambda e: (e, 0)),
        out_shape=jax.ShapeDtypeStruct((NSLOT, DP), jnp.int32),
        compiler_params=pltpu.CompilerParams(
            dimension_semantics=("parallel",)
        ),
    )(disp, w_in, w_out)


@functools.cache
def _make_combine():
    @functools.partial(
        pl.kernel,
        out_type=jax.ShapeDtypeStruct((N, DP), jnp.int32),
        mesh=_sc_mesh(),
        scratch_types=[
            pltpu.VMEM((CH,), jnp.int32),
            pltpu.VMEM((CH, DP), jnp.int32),
            pltpu.SemaphoreType.DMA,
        ],
    )
    def _combine(y_hbm, idx_hbm, out_hbm, idx_v, ybuf, sem):
        wid = lax.axis_index("s") * 2 + lax.axis_index("c")
        for j in range(NCH):
            pltpu.sync_copy(idx_hbm.at[wid * NCH + j], idx_v)
            pltpu.async_copy(y_hbm.at[idx_v], ybuf, sem).wait()
            pltpu.sync_copy(ybuf, out_hbm.at[pl.ds(wid * TPW + j * CH, CH)])

    return _combine


def _finish_body(g_ref, s_ref, o_ref):
    g = g_ref[...]
    s = s_ref[...].reshape(B, 1)
    o_ref[:, :DP] = _unpack_lo(g) * s
    o_ref[:, DP:] = _unpack_hi(g) * s


def _finish(gathered, scale):
    return pl.pallas_call(
        _finish_body,
        grid=(NB,),
        in_specs=[
            pl.BlockSpec((B, DP), lambda i: (i, 0)),
            pl.BlockSpec((1, 1, B), lambda i: (i, 0, 0)),
        ],
        out_specs=pl.BlockSpec((B, D), lambda i: (i, 0)),
        out_shape=jax.ShapeDtypeStruct((N, D), jnp.float32),
    )(gathered, scale)


def kernel(x, w_router, w_in, w_out):
    scat, gath, scale, xp = _router(x, w_router)
    scat = scat.reshape(NW * NCH, CH)
    gath = gath.reshape(NW * NCH, CH)
    disp = _make_dispatch()(xp, scat)
    y = _ffn(disp, w_in, w_out)
    gathered = _make_combine()(y, gath)
    return _finish(gathered, scale)


# speedup vs baseline: 1.0501x; 1.0067x over previous
import functools

import jax
import jax.numpy as jnp
from jax import lax
from jax.experimental import pallas as pl
from jax.experimental.pallas import tpu as pltpu
from jax.experimental.pallas import tpu_sc as plsc

N, D, E, DFF = 8192, 768, 64, 1536
DP = D // 2
CAP = 160
NSLOT = E * CAP
TRASH = NSLOT
B = 512
NB = N // B
NW = 32
TPW = N // NW
CH = 128
NCH = TPW // CH

_HI = -65536


def _pack_bf16_pair(a_f32, b_f32):
    ai = lax.bitcast_convert_type(a_f32, jnp.int32)
    bi = lax.bitcast_convert_type(b_f32, jnp.int32)
    pa = lax.shift_right_logical(ai + 0x8000, 16)
    pb = (bi + 0x8000) & _HI
    return pa | pb


def _unpack_lo(p):
    return lax.bitcast_convert_type(p << 16, jnp.float32)


def _unpack_hi(p):
    return lax.bitcast_convert_type(p & _HI, jnp.float32)


def _router_body(x_ref, wr_ref, scat_ref, gath_ref, scale_ref, xp_ref, counts_ref):
    step = pl.program_id(0)

    @pl.when(step == 0)
    def _():
        counts_ref[...] = jnp.zeros_like(counts_ref)

    x = x_ref[...]
    xp_ref[...] = _pack_bf16_pair(x[:, :DP], x[:, DP:])

    logitsT = lax.dot_general(
        wr_ref[...], x, (((0,), (1,)), ((), ())),
        preferred_element_type=jnp.float32,
    )
    m = jnp.max(logitsT, axis=0, keepdims=True)
    ssum = jnp.sum(jnp.exp(logitsT - m), axis=0, keepdims=True)
    gate = 1.0 / ssum

    row = lax.broadcasted_iota(jnp.int32, (E, B), 0)
    e_idx = jnp.min(jnp.where(logitsT == m, row, E), axis=0, keepdims=True)
    onehotT = (row == e_idx).astype(jnp.bfloat16)

    r_io = lax.broadcasted_iota(jnp.int32, (B, B), 0)
    c_io = lax.broadcasted_iota(jnp.int32, (B, B), 1)
    triu = (r_io <= c_io).astype(jnp.bfloat16)
    csumT = jnp.dot(onehotT, triu, preferred_element_type=jnp.float32)
    onehot_f = onehotT.astype(jnp.float32)
    rank_incl = jnp.sum(csumT * onehot_f, axis=0, keepdims=True)
    base = jnp.dot(counts_ref[...], onehot_f, preferred_element_type=jnp.float32)
    counts_ref[...] = counts_ref[...] + csumT[:, B - 1 : B].reshape(1, E)

    pos = base + rank_incl - 1.0
    keep = pos < CAP
    clipped = jnp.minimum(pos, CAP - 1.0).astype(jnp.int32)
    gath = e_idx * CAP + clipped
    scat = jnp.where(keep, gath, TRASH)

    scat_ref[...] = scat.reshape(1, 1, B)
    gath_ref[...] = gath.reshape(1, 1, B)
    scale_ref[...] = jnp.where(keep, gate, 0.0).reshape(1, 1, B)


def _router(x, w_router):
    return pl.pallas_call(
        _router_body,
        grid=(NB,),
        in_specs=[
            pl.BlockSpec((B, D), lambda i: (i, 0)),
            pl.BlockSpec((D, E), lambda i: (0, 0)),
        ],
        out_specs=[
            pl.BlockSpec((1, 1, B), lambda i: (i, 0, 0)),
            pl.BlockSpec((1, 1, B), lambda i: (i, 0, 0)),
            pl.BlockSpec((1, 1, B), lambda i: (i, 0, 0)),
            pl.BlockSpec((B, DP), lambda i: (i, 0)),
        ],
        out_shape=[
            jax.ShapeDtypeStruct((NB, 1, B), jnp.int32),
            jax.ShapeDtypeStruct((NB, 1, B), jnp.int32),
            jax.ShapeDtypeStruct((NB, 1, B), jnp.float32),
            jax.ShapeDtypeStruct((N, DP), jnp.int32),
        ],
        scratch_shapes=[pltpu.VMEM((1, E), jnp.float32)],
        compiler_params=pltpu.CompilerParams(
            dimension_semantics=("arbitrary",)
        ),
    )(x, w_router)


@functools.cache
def _sc_mesh():
    return plsc.VectorSubcoreMesh(
        core_axis_name="c", subcore_axis_name="s", num_cores=2, num_subcores=16
    )


@functools.cache
def _make_dispatch():
    @functools.partial(
        pl.kernel,
        out_type=jax.ShapeDtypeStruct((NSLOT + 8, DP), jnp.int32),
        mesh=_sc_mesh(),
        scratch_types=[
            pltpu.VMEM((NCH, CH), jnp.int32),
            pltpu.VMEM((CH, DP), jnp.int32),
            pltpu.VMEM((CH, DP), jnp.int32),
            pltpu.SemaphoreType.DMA,
            pltpu.SemaphoreType.DMA,
        ],
    )
    def _dispatch(xp_hbm, idx_hbm, disp_hbm, idx_v, xb0, xb1, sem0, sem1):
        wid = lax.axis_index("s") * 2 + lax.axis_index("c")
        pltpu.sync_copy(idx_hbm.at[pl.ds(wid * NCH, NCH)], idx_v)
        pltpu.sync_copy(xp_hbm.at[pl.ds(wid * TPW, CH)], xb0)
        cp0 = pltpu.async_copy(xb0, disp_hbm.at[idx_v.at[0]], sem0)
        pltpu.sync_copy(xp_hbm.at[pl.ds(wid * TPW + CH, CH)], xb1)
        cp1 = pltpu.async_copy(xb1, disp_hbm.at[idx_v.at[1]], sem1)
        cp0.wait()
        cp1.wait()

    return _dispatch


def _ffn_body(d_ref, wi_ref, wo_ref, y_ref):
    d = d_ref[...]
    da = _unpack_lo(d)
    db = _unpack_hi(d)
    wi = wi_ref[0]
    h = jnp.maximum(
        jnp.dot(da, wi[:DP], preferred_element_type=jnp.float32)
        + jnp.dot(db, wi[DP:], preferred_element_type=jnp.float32),
        0.0,
    )
    y = jnp.dot(h, wo_ref[0], preferred_element_type=jnp.float32)
    y_ref[...] = _pack_bf16_pair(y[:, :DP], y[:, DP:])


def _ffn(disp, w_in, w_out):
    return pl.pallas_call(
        _ffn_body,
        grid=(E,),
        in_specs=[
            pl.BlockSpec((CAP, DP), lambda e: (e, 0)),
            pl.BlockSpec((1, D, DFF), lambda e: (e, 0, 0)),
            pl.BlockSpec((1, DFF, D), lambda e: (e, 0, 0)),
        ],
        out_specs=pl.BlockSpec((CAP, DP), lambda e: (e, 0)),
        out_shape=jax.ShapeDtypeStruct((NSLOT, DP), jnp.int32),
        compiler_params=pltpu.CompilerParams(
            dimension_semantics=("parallel",)
        ),
    )(disp, w_in, w_out)


@functools.cache
def _make_combine():
    @functools.partial(
        pl.kernel,
        out_type=jax.ShapeDtypeStruct((N, DP), jnp.int32),
        mesh=_sc_mesh(),
        scratch_types=[
            pltpu.VMEM((NCH, CH), jnp.int32),
            pltpu.VMEM((CH, DP), jnp.int32),
            pltpu.VMEM((CH, DP), jnp.int32),
            pltpu.SemaphoreType.DMA,
            pltpu.SemaphoreType.DMA,
        ],
    )
    def _combine(y_hbm, idx_hbm, out_hbm, idx_v, yb0, yb1, sem0, sem1):
        wid = lax.axis_index("s") * 2 + lax.axis_index("c")
        pltpu.sync_copy(idx_hbm.at[pl.ds(wid * NCH, NCH)], idx_v)
        g0 = pltpu.async_copy(y_hbm.at[idx_v.at[0]], yb0, sem0)
        g1 = pltpu.async_copy(y_hbm.at[idx_v.at[1]], yb1, sem1)
        g0.wait()
        pltpu.sync_copy(yb0, out_hbm.at[pl.ds(wid * TPW, CH)])
        g1.wait()
        pltpu.sync_copy(yb1, out_hbm.at[pl.ds(wid * TPW + CH, CH)])

    return _combine


def _finish_body(g_ref, s_ref, o_ref):
    g = g_ref[...]
    s = s_ref[...].reshape(B, 1)
    o_ref[:, :DP] = _unpack_lo(g) * s
    o_ref[:, DP:] = _unpack_hi(g) * s


def _finish(gathered, scale):
    return pl.pallas_call(
        _finish_body,
        grid=(NB,),
        in_specs=[
            pl.BlockSpec((B, DP), lambda i: (i, 0)),
            pl.BlockSpec((1, 1, B), lambda i: (i, 0, 0)),
        ],
        out_specs=pl.BlockSpec((B, D), lambda i: (i, 0)),
        out_shape=jax.ShapeDtypeStruct((N, D), jnp.float32),
    )(gathered, scale)


def kernel(x, w_router, w_in, w_out):
    scat, gath, scale, xp = _router(x, w_router)
    scat = scat.reshape(NW * NCH, CH)
    gath = gath.reshape(NW * NCH, CH)
    disp = _make_dispatch()(xp, scat)
    y = _ffn(disp, w_in, w_out)
    gathered = _make_combine()(y, gath)
    return _finish(gathered, scale)


# speedup vs baseline: 1.0999x; 1.0474x over previous
import functools

import jax
import jax.numpy as jnp
from jax import lax
from jax.experimental import pallas as pl
from jax.experimental.pallas import tpu as pltpu
from jax.experimental.pallas import tpu_sc as plsc

N, D, E, DFF = 8192, 768, 64, 1536
DP = D // 2
CAP = 160
NSLOT = E * CAP
TRASH = NSLOT
B = 1024
NB = N // B
NW = 32
TPW = N // NW
CH = 128
NCH = TPW // CH

_HI = -65536


def _pack_bf16_pair(a_f32, b_f32):
    ai = lax.bitcast_convert_type(a_f32, jnp.int32)
    bi = lax.bitcast_convert_type(b_f32, jnp.int32)
    pa = lax.shift_right_logical(ai + 0x8000, 16)
    pb = (bi + 0x8000) & _HI
    return pa | pb


def _unpack_lo(p):
    return lax.bitcast_convert_type(p << 16, jnp.float32)


def _unpack_hi(p):
    return lax.bitcast_convert_type(p & _HI, jnp.float32)


def _router_body(x_ref, wr_ref, scat_ref, gath_ref, scale_ref, xp_ref, counts_ref):
    step = pl.program_id(0)

    @pl.when(step == 0)
    def _():
        counts_ref[...] = jnp.zeros_like(counts_ref)

    x = x_ref[...]
    xp_ref[...] = _pack_bf16_pair(x[:, :DP], x[:, DP:])

    logitsT = lax.dot_general(
        wr_ref[...], x, (((0,), (1,)), ((), ())),
        preferred_element_type=jnp.float32,
    )
    m = jnp.max(logitsT, axis=0, keepdims=True)
    ssum = jnp.sum(jnp.exp(logitsT - m), axis=0, keepdims=True)
    gate = 1.0 / ssum

    row = lax.broadcasted_iota(jnp.int32, (E, B), 0)
    e_idx = jnp.min(jnp.where(logitsT == m, row, E), axis=0, keepdims=True)
    onehotT = (row == e_idx).astype(jnp.bfloat16)

    r_io = lax.broadcasted_iota(jnp.int32, (B, B), 0)
    c_io = lax.broadcasted_iota(jnp.int32, (B, B), 1)
    triu = (r_io <= c_io).astype(jnp.bfloat16)
    csumT = jnp.dot(onehotT, triu, preferred_element_type=jnp.float32)
    onehot_f = onehotT.astype(jnp.float32)
    rank_incl = jnp.sum(csumT * onehot_f, axis=0, keepdims=True)
    base = jnp.dot(counts_ref[...], onehot_f, preferred_element_type=jnp.float32)
    counts_ref[...] = counts_ref[...] + csumT[:, B - 1 : B].reshape(1, E)

    pos = base + rank_incl - 1.0
    keep = pos < CAP
    clipped = jnp.minimum(pos, CAP - 1.0).astype(jnp.int32)
    gath = e_idx * CAP + clipped
    scat = jnp.where(keep, gath, TRASH)

    scat_ref[...] = scat.reshape(1, 1, B)
    gath_ref[...] = gath.reshape(1, 1, B)
    scale_ref[...] = jnp.where(keep, gate, 0.0).reshape(1, 1, B)


def _router(x, w_router):
    return pl.pallas_call(
        _router_body,
        grid=(NB,),
        in_specs=[
            pl.BlockSpec((B, D), lambda i: (i, 0)),
            pl.BlockSpec((D, E), lambda i: (0, 0)),
        ],
        out_specs=[
            pl.BlockSpec((1, 1, B), lambda i: (i, 0, 0)),
            pl.BlockSpec((1, 1, B), lambda i: (i, 0, 0)),
            pl.BlockSpec((1, 1, B), lambda i: (i, 0, 0)),
            pl.BlockSpec((B, DP), lambda i: (i, 0)),
        ],
        out_shape=[
            jax.ShapeDtypeStruct((NB, 1, B), jnp.int32),
            jax.ShapeDtypeStruct((NB, 1, B), jnp.int32),
            jax.ShapeDtypeStruct((NB, 1, B), jnp.float32),
            jax.ShapeDtypeStruct((N, DP), jnp.int32),
        ],
        scratch_shapes=[pltpu.VMEM((1, E), jnp.float32)],
        compiler_params=pltpu.CompilerParams(
            dimension_semantics=("arbitrary",)
        ),
    )(x, w_router)


@functools.cache
def _sc_mesh():
    return plsc.VectorSubcoreMesh(
        core_axis_name="c", subcore_axis_name="s", num_cores=2, num_subcores=16
    )


@functools.cache
def _make_dispatch():
    @functools.partial(
        pl.kernel,
        out_type=jax.ShapeDtypeStruct((NSLOT + 8, DP), jnp.int32),
        mesh=_sc_mesh(),
        scratch_types=[
            pltpu.VMEM((NCH, CH), jnp.int32),
            pltpu.VMEM((CH, DP), jnp.int32),
            pltpu.VMEM((CH, DP), jnp.int32),
            pltpu.SemaphoreType.DMA,
            pltpu.SemaphoreType.DMA,
        ],
    )
    def _dispatch(xp_hbm, idx_hbm, disp_hbm, idx_v, xb0, xb1, sem0, sem1):
        wid = lax.axis_index("s") * 2 + lax.axis_index("c")
        pltpu.sync_copy(idx_hbm.at[pl.ds(wid * NCH, NCH)], idx_v)
        pltpu.sync_copy(xp_hbm.at[pl.ds(wid * TPW, CH)], xb0)
        cp0 = pltpu.async_copy(xb0, disp_hbm.at[idx_v.at[0]], sem0)
        pltpu.sync_copy(xp_hbm.at[pl.ds(wid * TPW + CH, CH)], xb1)
        cp1 = pltpu.async_copy(xb1, disp_hbm.at[idx_v.at[1]], sem1)
        cp0.wait()
        cp1.wait()

    return _dispatch


def _ffn_body(d_ref, wi_ref, wo_ref, y_ref):
    d = d_ref[...]
    da = _unpack_lo(d)
    db = _unpack_hi(d)
    wi = wi_ref[0]
    h = jnp.maximum(
        jnp.dot(da, wi[:DP], preferred_element_type=jnp.float32)
        + jnp.dot(db, wi[DP:], preferred_element_type=jnp.float32),
        0.0,
    )
    y = jnp.dot(h, wo_ref[0], preferred_element_type=jnp.float32)
    y_ref[...] = _pack_bf16_pair(y[:, :DP], y[:, DP:])


def _ffn(disp, w_in, w_out):
    return pl.pallas_call(
        _ffn_body,
        grid=(E,),
        in_specs=[
            pl.BlockSpec((CAP, DP), lambda e: (e, 0)),
            pl.BlockSpec((1, D, DFF), lambda e: (e, 0, 0)),
            pl.BlockSpec((1, DFF, D), lambda e: (e, 0, 0)),
        ],
        out_specs=pl.BlockSpec((CAP, DP), lambda e: (e, 0)),
        out_shape=jax.ShapeDtypeStruct((NSLOT, DP), jnp.int32),
        compiler_params=pltpu.CompilerParams(
            dimension_semantics=("parallel",)
        ),
    )(disp, w_in, w_out)


@functools.cache
def _make_combine():
    @functools.partial(
        pl.kernel,
        out_type=jax.ShapeDtypeStruct((N, DP), jnp.int32),
        mesh=_sc_mesh(),
        scratch_types=[
            pltpu.VMEM((NCH, CH), jnp.int32),
            pltpu.VMEM((CH, DP), jnp.int32),
            pltpu.VMEM((CH, DP), jnp.int32),
            pltpu.SemaphoreType.DMA,
            pltpu.SemaphoreType.DMA,
        ],
    )
    def _combine(y_hbm, idx_hbm, out_hbm, idx_v, yb0, yb1, sem0, sem1):
        wid = lax.axis_index("s") * 2 + lax.axis_index("c")
        pltpu.sync_copy(idx_hbm.at[pl.ds(wid * NCH, NCH)], idx_v)
        g0 = pltpu.async_copy(y_hbm.at[idx_v.at[0]], yb0, sem0)
        g1 = pltpu.async_copy(y_hbm.at[idx_v.at[1]], yb1, sem1)
        g0.wait()
        pltpu.sync_copy(yb0, out_hbm.at[pl.ds(wid * TPW, CH)])
        g1.wait()
        pltpu.sync_copy(yb1, out_hbm.at[pl.ds(wid * TPW + CH, CH)])

    return _combine


def _finish_body(g_ref, s_ref, o_ref):
    g = g_ref[...]
    s = s_ref[...].reshape(B, 1)
    o_ref[:, :DP] = _unpack_lo(g) * s
    o_ref[:, DP:] = _unpack_hi(g) * s


def _finish(gathered, scale):
    return pl.pallas_call(
        _finish_body,
        grid=(NB,),
        in_specs=[
            pl.BlockSpec((B, DP), lambda i: (i, 0)),
            pl.BlockSpec((1, 1, B), lambda i: (i, 0, 0)),
        ],
        out_specs=pl.BlockSpec((B, D), lambda i: (i, 0)),
        out_shape=jax.ShapeDtypeStruct((N, D), jnp.float32),
    )(gathered, scale)


def kernel(x, w_router, w_in, w_out):
    scat, gath, scale, xp = _router(x, w_router)
    scat = scat.reshape(NW * NCH, CH)
    gath = gath.reshape(NW * NCH, CH)
    disp = _make_dispatch()(xp, scat)
    y = _ffn(disp, w_in, w_out)
    gathered = _make_combine()(y, gath)
    return _finish(gathered, scale)


# speedup vs baseline: 1.1066x; 1.0061x over previous
import functools

import jax
import jax.numpy as jnp
from jax import lax
from jax.experimental import pallas as pl
from jax.experimental.pallas import tpu as pltpu
from jax.experimental.pallas import tpu_sc as plsc

N, D, E, DFF = 8192, 768, 64, 1536
DP = D // 2
CAP = 160
NSLOT = E * CAP
TRASH = NSLOT
B = 2048
NB = N // B
NW = 32
TPW = N // NW
CH = 128
NCH = TPW // CH

_HI = -65536


def _pack_bf16_pair(a_f32, b_f32):
    ai = lax.bitcast_convert_type(a_f32, jnp.int32)
    bi = lax.bitcast_convert_type(b_f32, jnp.int32)
    pa = lax.shift_right_logical(ai + 0x8000, 16)
    pb = (bi + 0x8000) & _HI
    return pa | pb


def _unpack_lo(p):
    return lax.bitcast_convert_type(p << 16, jnp.float32)


def _unpack_hi(p):
    return lax.bitcast_convert_type(p & _HI, jnp.float32)


def _router_body(x_ref, wr_ref, scat_ref, gath_ref, scale_ref, xp_ref, counts_ref):
    step = pl.program_id(0)

    @pl.when(step == 0)
    def _():
        counts_ref[...] = jnp.zeros_like(counts_ref)

    x = x_ref[...]
    xp_ref[...] = _pack_bf16_pair(x[:, :DP], x[:, DP:])

    logitsT = lax.dot_general(
        wr_ref[...], x, (((0,), (1,)), ((), ())),
        preferred_element_type=jnp.float32,
    )
    m = jnp.max(logitsT, axis=0, keepdims=True)
    ssum = jnp.sum(jnp.exp(logitsT - m), axis=0, keepdims=True)
    gate = 1.0 / ssum

    row = lax.broadcasted_iota(jnp.int32, (E, B), 0)
    e_idx = jnp.min(jnp.where(logitsT == m, row, E), axis=0, keepdims=True)
    onehotT = (row == e_idx).astype(jnp.bfloat16)

    r_io = lax.broadcasted_iota(jnp.int32, (B, B), 0)
    c_io = lax.broadcasted_iota(jnp.int32, (B, B), 1)
    triu = (r_io <= c_io).astype(jnp.bfloat16)
    csumT = jnp.dot(onehotT, triu, preferred_element_type=jnp.float32)
    onehot_f = onehotT.astype(jnp.float32)
    rank_incl = jnp.sum(csumT * onehot_f, axis=0, keepdims=True)
    base = jnp.dot(counts_ref[...], onehot_f, preferred_element_type=jnp.float32)
    counts_ref[...] = counts_ref[...] + csumT[:, B - 1 : B].reshape(1, E)

    pos = base + rank_incl - 1.0
    keep = pos < CAP
    clipped = jnp.minimum(pos, CAP - 1.0).astype(jnp.int32)
    gath = e_idx * CAP + clipped
    scat = jnp.where(keep, gath, TRASH)

    scat_ref[...] = scat.reshape(1, 1, B)
    gath_ref[...] = gath.reshape(1, 1, B)
    scale_ref[...] = jnp.where(keep, gate, 0.0).reshape(1, 1, B)


def _router(x, w_router):
    return pl.pallas_call(
        _router_body,
        grid=(NB,),
        in_specs=[
            pl.BlockSpec((B, D), lambda i: (i, 0)),
            pl.BlockSpec((D, E), lambda i: (0, 0)),
        ],
        out_specs=[
            pl.BlockSpec((1, 1, B), lambda i: (i, 0, 0)),
            pl.BlockSpec((1, 1, B), lambda i: (i, 0, 0)),
            pl.BlockSpec((1, 1, B), lambda i: (i, 0, 0)),
            pl.BlockSpec((B, DP), lambda i: (i, 0)),
        ],
        out_shape=[
            jax.ShapeDtypeStruct((NB, 1, B), jnp.int32),
            jax.ShapeDtypeStruct((NB, 1, B), jnp.int32),
            jax.ShapeDtypeStruct((NB, 1, B), jnp.float32),
            jax.ShapeDtypeStruct((N, DP), jnp.int32),
        ],
        scratch_shapes=[pltpu.VMEM((1, E), jnp.float32)],
        compiler_params=pltpu.CompilerParams(
            dimension_semantics=("arbitrary",)
        ),
    )(x, w_router)


@functools.cache
def _sc_mesh():
    return plsc.VectorSubcoreMesh(
        core_axis_name="c", subcore_axis_name="s", num_cores=2, num_subcores=16
    )


@functools.cache
def _make_dispatch():
    @functools.partial(
        pl.kernel,
        out_type=jax.ShapeDtypeStruct((NSLOT + 8, DP), jnp.int32),
        mesh=_sc_mesh(),
        scratch_types=[
            pltpu.VMEM((NCH, CH), jnp.int32),
            pltpu.VMEM((CH, DP), jnp.int32),
            pltpu.VMEM((CH, DP), jnp.int32),
            pltpu.SemaphoreType.DMA,
            pltpu.SemaphoreType.DMA,
        ],
    )
    def _dispatch(xp_hbm, idx_hbm, disp_hbm, idx_v, xb0, xb1, sem0, sem1):
        wid = lax.axis_index("s") * 2 + lax.axis_index("c")
        pltpu.sync_copy(idx_hbm.at[pl.ds(wid * NCH, NCH)], idx_v)
        pltpu.sync_copy(xp_hbm.at[pl.ds(wid * TPW, CH)], xb0)
        cp0 = pltpu.async_copy(xb0, disp_hbm.at[idx_v.at[0]], sem0)
        pltpu.sync_copy(xp_hbm.at[pl.ds(wid * TPW + CH, CH)], xb1)
        cp1 = pltpu.async_copy(xb1, disp_hbm.at[idx_v.at[1]], sem1)
        cp0.wait()
        cp1.wait()

    return _dispatch


def _ffn_body(d_ref, wi_ref, wo_ref, y_ref):
    d = d_ref[...]
    da = _unpack_lo(d)
    db = _unpack_hi(d)
    wi = wi_ref[0]
    h = jnp.maximum(
        jnp.dot(da, wi[:DP], preferred_element_type=jnp.float32)
        + jnp.dot(db, wi[DP:], preferred_element_type=jnp.float32),
        0.0,
    )
    y = jnp.dot(h, wo_ref[0], preferred_element_type=jnp.float32)
    y_ref[...] = _pack_bf16_pair(y[:, :DP], y[:, DP:])


def _ffn(disp, w_in, w_out):
    return pl.pallas_call(
        _ffn_body,
        grid=(E,),
        in_specs=[
            pl.BlockSpec((CAP, DP), lambda e: (e, 0)),
            pl.BlockSpec((1, D, DFF), lambda e: (e, 0, 0)),
            pl.BlockSpec((1, DFF, D), lambda e: (e, 0, 0)),
        ],
        out_specs=pl.BlockSpec((CAP, DP), lambda e: (e, 0)),
        out_shape=jax.ShapeDtypeStruct((NSLOT, DP), jnp.int32),
        compiler_params=pltpu.CompilerParams(
            dimension_semantics=("parallel",)
        ),
    )(disp, w_in, w_out)


@functools.cache
def _make_combine():
    @functools.partial(
        pl.kernel,
        out_type=jax.ShapeDtypeStruct((N, DP), jnp.int32),
        mesh=_sc_mesh(),
        scratch_types=[
            pltpu.VMEM((NCH, CH), jnp.int32),
            pltpu.VMEM((CH, DP), jnp.int32),
            pltpu.VMEM((CH, DP), jnp.int32),
            pltpu.SemaphoreType.DMA,
            pltpu.SemaphoreType.DMA,
        ],
    )
    def _combine(y_hbm, idx_hbm, out_hbm, idx_v, yb0, yb1, sem0, sem1):
        wid = lax.axis_index("s") * 2 + lax.axis_index("c")
        pltpu.sync_copy(idx_hbm.at[pl.ds(wid * NCH, NCH)], idx_v)
        g0 = pltpu.async_copy(y_hbm.at[idx_v.at[0]], yb0, sem0)
        g1 = pltpu.async_copy(y_hbm.at[idx_v.at[1]], yb1, sem1)
        g0.wait()
        pltpu.sync_copy(yb0, out_hbm.at[pl.ds(wid * TPW, CH)])
        g1.wait()
        pltpu.sync_copy(yb1, out_hbm.at[pl.ds(wid * TPW + CH, CH)])

    return _combine


def _finish_body(g_ref, s_ref, o_ref):
    g = g_ref[...]
    s = s_ref[...].reshape(B, 1)
    o_ref[:, :DP] = _unpack_lo(g) * s
    o_ref[:, DP:] = _unpack_hi(g) * s


def _finish(gathered, scale):
    return pl.pallas_call(
        _finish_body,
        grid=(NB,),
        in_specs=[
            pl.BlockSpec((B, DP), lambda i: (i, 0)),
            pl.BlockSpec((1, 1, B), lambda i: (i, 0, 0)),
        ],
        out_specs=pl.BlockSpec((B, D), lambda i: (i, 0)),
        out_shape=jax.ShapeDtypeStruct((N, D), jnp.float32),
    )(gathered, scale)


def kernel(x, w_router, w_in, w_out):
    scat, gath, scale, xp = _router(x, w_router)
    scat = scat.reshape(NW * NCH, CH)
    gath = gath.reshape(NW * NCH, CH)
    disp = _make_dispatch()(xp, scat)
    y = _ffn(disp, w_in, w_out)
    gathered = _make_combine()(y, gath)
    return _finish(gathered, scale)
